# Initial kernel scaffold; baseline (speedup 1.0000x reference)
import jax
import jax.numpy as jnp
from jax.experimental import pallas as pl


def kernel(x, W1, b1, g1, be1, W2, b2, g2, be2, W3, b3, g3, be3, Wh, bh, indices):
    raise NotImplementedError("write your pallas kernel here")


# speedup vs baseline: 1.4928x; 1.4928x over previous
import functools

import jax
import jax.numpy as jnp
from jax.experimental import pallas as pl
from jax.experimental.pallas import tpu as pltpu

_BN_EPS = 1e-5


def _mlp_kernel(x_ref, W1_ref, b1_ref, g1_ref, be1_ref,
                W2_ref, b2_ref, g2_ref, be2_ref,
                W3_ref, b3_ref, g3_ref, be3_ref,
                Wh_ref, bh_ref,
                out_ref,
                s1, ss1, a1, c1,
                s2, ss2, a2, c2,
                s3, ss3, a3, c3,
                y2_buf, y3_buf,
                *, tb, inv_b):
    p = pl.program_id(0)
    t = pl.program_id(1)

    def finalize(s, ss, g_ref, be_ref, a, c):
        mu = s[...] * inv_b
        var = ss[...] * inv_b - mu * mu
        istd = jax.lax.rsqrt(var + _BN_EPS)
        ai = g_ref[...] * istd
        a[...] = ai
        c[...] = be_ref[...] - mu * ai

    def layer1():
        xt = x_ref[...]
        return jnp.dot(xt, W1_ref[...],
                       preferred_element_type=jnp.float32) + b1_ref[...]

    @pl.when(p == 0)
    def _pass0():
        @pl.when(t == 0)
        def _():
            s1[...] = jnp.zeros_like(s1)
            ss1[...] = jnp.zeros_like(ss1)
        y1 = layer1()
        s1[...] += jnp.sum(y1, axis=0, keepdims=True)
        ss1[...] += jnp.sum(y1 * y1, axis=0, keepdims=True)

    @pl.when(p == 1)
    def _pass1():
        @pl.when(t == 0)
        def _():
            finalize(s1, ss1, g1_ref, be1_ref, a1, c1)
            s2[...] = jnp.zeros_like(s2)
            ss2[...] = jnp.zeros_like(ss2)
        y1 = layer1()
        h1 = jnp.maximum(y1 * a1[...] + c1[...], 0.0)
        y2 = jnp.dot(h1, W2_ref[...],
                     preferred_element_type=jnp.float32) + b2_ref[...]
        y2_buf[pl.ds(t * tb, tb), :] = y2
        s2[...] += jnp.sum(y2, axis=0, keepdims=True)
        ss2[...] += jnp.sum(y2 * y2, axis=0, keepdims=True)

    @pl.when(p == 2)
    def _pass2():
        @pl.when(t == 0)
        def _():
            finalize(s2, ss2, g2_ref, be2_ref, a2, c2)
            s3[...] = jnp.zeros_like(s3)
            ss3[...] = jnp.zeros_like(ss3)
        y2 = y2_buf[pl.ds(t * tb, tb), :]
        h2 = jnp.maximum(y2 * a2[...] + c2[...], 0.0)
        y3 = jnp.dot(h2, W3_ref[...],
                     preferred_element_type=jnp.float32) + b3_ref[...]
        y3_buf[pl.ds(t * tb, tb), :] = y3
        s3[...] += jnp.sum(y3, axis=0, keepdims=True)
        ss3[...] += jnp.sum(y3 * y3, axis=0, keepdims=True)

    @pl.when(p == 3)
    def _pass3():
        @pl.when(t == 0)
        def _():
            finalize(s3, ss3, g3_ref, be3_ref, a3, c3)
        y3 = y3_buf[pl.ds(t * tb, tb), :]
        h3 = jnp.maximum(y3 * a3[...] + c3[...], 0.0)
        z = jnp.dot(h3, Wh_ref[...],
                    preferred_element_type=jnp.float32) + bh_ref[...]
        n = jnp.sqrt(jnp.sum(z * z, axis=1, keepdims=True))
        out_ref[...] = z / jnp.maximum(n, 1e-12)


def _rup(n, m=128):
    return (n + m - 1) // m * m


def kernel(x, W1, b1, g1, be1, W2, b2, g2, be2, W3, b3, g3, be3, Wh, bh,
           indices):
    del indices
    B, in_dim = x.shape
    d1, d2, d3, feat = W1.shape[1], W2.shape[1], W3.shape[1], Wh.shape[1]
    d1p, d2p, d3p = _rup(d1), _rup(d2), _rup(d3)

    def pad_w(w, r, c):
        return jnp.pad(w, ((0, r - w.shape[0]), (0, c - w.shape[1])))

    def pad_v(v, n):
        return jnp.pad(v, (0, n - v.shape[0])).reshape(1, n)

    W1p, b1p = pad_w(W1, in_dim, d1p), pad_v(b1, d1p)
    g1p, be1p = pad_v(g1, d1p), pad_v(be1, d1p)
    W2p, b2p = pad_w(W2, d1p, d2p), pad_v(b2, d2p)
    g2p, be2p = pad_v(g2, d2p), pad_v(be2, d2p)
    W3p, b3p = pad_w(W3, d2p, d3p), pad_v(b3, d3p)
    g3p, be3p = pad_v(g3, d3p), pad_v(be3, d3p)
    Whp, bhp = pad_w(Wh, d3p, feat), pad_v(bh, feat)

    tb = 1024
    T = B // tb

    def const_spec(shape):
        return pl.BlockSpec(shape, lambda p, t: (0, 0))

    in_specs = [
        pl.BlockSpec((tb, in_dim), lambda p, t: (jnp.where(p < 2, t, 0), 0)),
        const_spec((in_dim, d1p)), const_spec((1, d1p)),
        const_spec((1, d1p)), const_spec((1, d1p)),
        const_spec((d1p, d2p)), const_spec((1, d2p)),
        const_spec((1, d2p)), const_spec((1, d2p)),
        const_spec((d2p, d3p)), const_spec((1, d3p)),
        const_spec((1, d3p)), const_spec((1, d3p)),
        const_spec((d3p, feat)), const_spec((1, feat)),
    ]
    out_spec = pl.BlockSpec((tb, feat),
                            lambda p, t: (jnp.where(p == 3, t, 0), 0))
    scratch_shapes = [
        pltpu.VMEM((1, d1p), jnp.float32), pltpu.VMEM((1, d1p), jnp.float32),
        pltpu.VMEM((1, d1p), jnp.float32), pltpu.VMEM((1, d1p), jnp.float32),
        pltpu.VMEM((1, d2p), jnp.float32), pltpu.VMEM((1, d2p), jnp.float32),
        pltpu.VMEM((1, d2p), jnp.float32), pltpu.VMEM((1, d2p), jnp.float32),
        pltpu.VMEM((1, d3p), jnp.float32), pltpu.VMEM((1, d3p), jnp.float32),
        pltpu.VMEM((1, d3p), jnp.float32), pltpu.VMEM((1, d3p), jnp.float32),
        pltpu.VMEM((B, d2p), jnp.float32),
        pltpu.VMEM((B, d3p), jnp.float32),
    ]

    out = pl.pallas_call(
        functools.partial(_mlp_kernel, tb=tb, inv_b=1.0 / B),
        grid=(4, T),
        in_specs=in_specs,
        out_specs=out_spec,
        out_shape=jax.ShapeDtypeStruct((B, feat), jnp.float32),
        scratch_shapes=scratch_shapes,
    )(x, W1p, b1p, g1p, be1p, W2p, b2p, g2p, be2p,
      W3p, b3p, g3p, be3p, Whp, bhp)
    return out
